# Initial kernel scaffold; baseline (speedup 1.0000x reference)
import jax
import jax.numpy as jnp
from jax.experimental import pallas as pl


def kernel(attr_vals, obs_idx, W_attr, b_attr, q_attn, W_ent, action_table):
    raise NotImplementedError("write your pallas kernel here")


# speedup vs baseline: 2.8438x; 2.8438x over previous
import functools

import jax
import jax.numpy as jnp
from jax import lax
from jax.experimental import pallas as pl
from jax.experimental.pallas import tpu as pltpu
from jax.experimental.pallas import tpu_sc as plsc

N = 16384
A = 26
H = 64
NOBJ = 1024
K = 20
NACT = 8

AH = A * H
BN = 512
NW = 32
OBJ_PER_W = NOBJ // NW
IDX_PER_W = OBJ_PER_W * K
CHUNK = 128
NCHUNK = IDX_PER_W // CHUNK


def _entity_tile(vals_ref, w2_ref, b2_ref, q2_ref, rep_ref, rept_ref,
                 wa_ref, ba_ref, went_ref, el_ref, el2_ref):
    vals = vals_ref[...]
    vals_rep = jnp.dot(vals, rep_ref[...], preferred_element_type=jnp.float32)
    emb = vals_rep * w2_ref[...] + b2_ref[...]
    tq = jnp.tanh(emb) * q2_ref[...]
    scores = jnp.dot(tq, rept_ref[...], preferred_element_type=jnp.float32)
    m = jnp.max(scores, axis=-1, keepdims=True)
    e = jnp.exp(scores - m)
    w = e / jnp.sum(e, axis=-1, keepdims=True)
    el = (jnp.dot(w * vals, wa_ref[...], preferred_element_type=jnp.float32)
          + jnp.dot(w, ba_ref[...], preferred_element_type=jnp.float32))
    el_ref[...] = el
    el2 = jnp.dot(el, went_ref[...], preferred_element_type=jnp.float32)
    el2_ref[...] = jnp.concatenate([el2, jnp.zeros_like(el2)], axis=1)


def _entity_lookup(attr_vals, W_attr, b_attr, q_attn, W_ent):
    w2 = W_attr.reshape(1, AH)
    b2 = b_attr.reshape(1, AH)
    q2 = jnp.tile(q_attn, (A,)).reshape(1, AH)
    eye = jnp.eye(A, dtype=jnp.float32)
    rep = jnp.kron(eye, jnp.ones((1, H), jnp.float32))
    rept = jnp.kron(eye, jnp.ones((H, 1), jnp.float32))
    grid = (N // BN,)
    full = lambda *shape: pl.BlockSpec(shape, lambda i: (0,) * len(shape))
    return pl.pallas_call(
        _entity_tile,
        grid=grid,
        in_specs=[
            pl.BlockSpec((BN, A), lambda i: (i, 0)),
            full(1, AH), full(1, AH), full(1, AH),
            full(A, AH), full(AH, A),
            full(A, H), full(A, H), full(H, H),
        ],
        out_specs=[
            pl.BlockSpec((BN, H), lambda i: (i, 0)),
            pl.BlockSpec((BN, 2 * H), lambda i: (i, 0)),
        ],
        out_shape=[
            jax.ShapeDtypeStruct((N, H), jnp.float32),
            jax.ShapeDtypeStruct((N, 2 * H), jnp.float32),
        ],
    )(attr_vals, w2, b2, q2, rep, rept, W_attr, b_attr, W_ent)


def _gather_body(el2_hbm, idx_hbm, out_hbm, idx_v, rows_v, pooled_v, sem):
    wid = lax.axis_index("s") * 2 + lax.axis_index("c")
    pltpu.sync_copy(idx_hbm.at[wid], idx_v)
    copies = [
        pltpu.async_copy(el2_hbm.at[idx_v.at[j]],
                         rows_v.at[pl.ds(j * CHUNK, CHUNK)], sem)
        for j in range(NCHUNK)
    ]
    for c in copies:
        c.wait()

    inv_k = jnp.float32(1.0 / K)

    def body(o, carry):
        for c in range(H // 16):
            acc = rows_v[o * K, pl.ds(c * 16, 16)]
            for k in range(1, K):
                acc = acc + rows_v[o * K + k, pl.ds(c * 16, 16)]
            y = acc * inv_k
            t = 1.0 - 2.0 / (jnp.exp(2.0 * y) + 1.0)
            pooled_v[o, pl.ds(c * 16, 16)] = t
        return carry

    lax.fori_loop(0, OBJ_PER_W, body, 0, unroll=False)
    pltpu.sync_copy(pooled_v, out_hbm.at[pl.ds(wid * OBJ_PER_W, OBJ_PER_W)])


def _gather_pool(el2, idx2d):
    mesh = plsc.VectorSubcoreMesh(core_axis_name="c", subcore_axis_name="s")
    fn = functools.partial(
        pl.kernel,
        mesh=mesh,
        out_type=jax.ShapeDtypeStruct((NOBJ, H), jnp.float32),
        scratch_types=[
            pltpu.VMEM((NCHUNK, CHUNK), jnp.int32),
            pltpu.VMEM((IDX_PER_W, 2 * H), jnp.float32),
            pltpu.VMEM((OBJ_PER_W, H), jnp.float32),
            pltpu.SemaphoreType.DMA,
        ],
    )(_gather_body)
    return fn(el2, idx2d)


def kernel(attr_vals, obs_idx, W_attr, b_attr, q_attn, W_ent, action_table):
    el, el2 = _entity_lookup(attr_vals, W_attr, b_attr, q_attn, W_ent)
    idx3d = obs_idx.astype(jnp.int32).reshape(NW, NCHUNK, CHUNK)
    obs = _gather_pool(el2, idx3d)
    pad = jnp.zeros((1, H), el.dtype)
    el_full = jnp.concatenate([el, action_table, pad], axis=0)
    return obs, el_full


# speedup vs baseline: 2.9180x; 1.0261x over previous
import functools

import jax
import jax.numpy as jnp
from jax import lax
from jax.experimental import pallas as pl
from jax.experimental.pallas import tpu as pltpu
from jax.experimental.pallas import tpu_sc as plsc

N = 16384
A = 26
H = 64
NOBJ = 1024
K = 20
NACT = 8

AH = A * H
BN = 512
GN = N // BN
NFULL = N + NACT + 1
NW = 32
OBJ_PER_W = NOBJ // NW
IDX_PER_W = OBJ_PER_W * K
CHUNK = 128
NCHUNK = IDX_PER_W // CHUNK


def _entity_tile(vals_ref, rwb_ref, rtq_ref, wa_ref, ba_ref, went_ref,
                 atn_ref, out_ref, el2_ref):
    i = pl.program_id(0)
    vals = vals_ref[...]
    vals1 = jnp.concatenate(
        [vals, jnp.ones((BN, 1), jnp.float32)], axis=1)
    emb = jnp.dot(vals1, rwb_ref[...], preferred_element_type=jnp.float32)
    scores = jnp.dot(jnp.tanh(emb), rtq_ref[...],
                     preferred_element_type=jnp.float32)
    m = jnp.max(scores, axis=-1, keepdims=True)
    e = jnp.exp(scores - m)
    w = e / jnp.sum(e, axis=-1, keepdims=True)
    el = (jnp.dot(w * vals, wa_ref[...], preferred_element_type=jnp.float32)
          + jnp.dot(w, ba_ref[...], preferred_element_type=jnp.float32))
    out_ref[...] = el
    el2 = jnp.dot(el, went_ref[...], preferred_element_type=jnp.float32)
    el2_ref[...] = jnp.concatenate([el2, jnp.zeros_like(el2)], axis=1)

    @pl.when(i == GN)
    def _():
        out_ref[0:16, :] = atn_ref[...]


def _entity_lookup(attr_vals, W_attr, b_attr, q_attn, W_ent, action_table):
    w2 = W_attr.reshape(1, AH)
    b2 = b_attr.reshape(1, AH)
    eye = jnp.eye(A, dtype=jnp.float32)
    rep = jnp.kron(eye, jnp.ones((1, H), jnp.float32))
    rwb = jnp.concatenate([rep * w2, b2], axis=0)
    rtq = (jnp.kron(eye, jnp.ones((H, 1), jnp.float32))
           * jnp.tile(q_attn, (A,))[:, None])
    atn16 = jnp.concatenate(
        [action_table, jnp.zeros((16 - NACT, H), jnp.float32)], axis=0)
    clamp = lambda i: (jnp.minimum(i, GN - 1), 0)
    full = lambda *shape: pl.BlockSpec(shape, lambda i: (0,) * len(shape))
    return pl.pallas_call(
        _entity_tile,
        grid=(GN + 1,),
        in_specs=[
            pl.BlockSpec((BN, A), clamp),
            full(A + 1, AH), full(AH, A),
            full(A, H), full(A, H), full(H, H), full(16, H),
        ],
        out_specs=[
            pl.BlockSpec((BN, H), lambda i: (i, 0)),
            pl.BlockSpec((BN, 2 * H), clamp),
        ],
        out_shape=[
            jax.ShapeDtypeStruct((NFULL, H), jnp.float32),
            jax.ShapeDtypeStruct((N, 2 * H), jnp.float32),
        ],
    )(attr_vals, rwb, rtq, W_attr, b_attr, W_ent, atn16)


def _gather_body(el2_hbm, idx_hbm, out_hbm, idx_v, rows_v, pooled_v, sem):
    wid = lax.axis_index("s") * 2 + lax.axis_index("c")
    pltpu.sync_copy(idx_hbm.at[wid], idx_v)
    copies = [
        pltpu.async_copy(el2_hbm.at[idx_v.at[j]],
                         rows_v.at[pl.ds(j * CHUNK, CHUNK)], sem)
        for j in range(NCHUNK)
    ]
    for c in copies:
        c.wait()

    inv_k = jnp.float32(1.0 / K)

    def body(o, carry):
        for c in range(H // 16):
            acc = rows_v[o * K, pl.ds(c * 16, 16)]
            for k in range(1, K):
                acc = acc + rows_v[o * K + k, pl.ds(c * 16, 16)]
            y = acc * inv_k
            t = 1.0 - 2.0 / (jnp.exp(2.0 * y) + 1.0)
            pooled_v[o, pl.ds(c * 16, 16)] = t
        return carry

    lax.fori_loop(0, OBJ_PER_W, body, 0, unroll=False)
    pltpu.sync_copy(pooled_v, out_hbm.at[pl.ds(wid * OBJ_PER_W, OBJ_PER_W)])


def _gather_pool(el2, idx3d):
    mesh = plsc.VectorSubcoreMesh(core_axis_name="c", subcore_axis_name="s")
    fn = functools.partial(
        pl.kernel,
        mesh=mesh,
        out_type=jax.ShapeDtypeStruct((NOBJ, H), jnp.float32),
        scratch_types=[
            pltpu.VMEM((NCHUNK, CHUNK), jnp.int32),
            pltpu.VMEM((IDX_PER_W, 2 * H), jnp.float32),
            pltpu.VMEM((OBJ_PER_W, H), jnp.float32),
            pltpu.SemaphoreType.DMA,
        ],
    )(_gather_body)
    return fn(el2, idx3d)


def kernel(attr_vals, obs_idx, W_attr, b_attr, q_attn, W_ent, action_table):
    el_full, el2 = _entity_lookup(attr_vals, W_attr, b_attr, q_attn,
                                  W_ent, action_table)
    idx3d = obs_idx.astype(jnp.int32).reshape(NW, NCHUNK, CHUNK)
    obs = _gather_pool(el2, idx3d)
    return obs, el_full


# speedup vs baseline: 3.2601x; 1.1173x over previous
import functools

import jax
import jax.numpy as jnp
from jax import lax
from jax.experimental import pallas as pl
from jax.experimental.pallas import tpu as pltpu
from jax.experimental.pallas import tpu_sc as plsc

N = 16384
A = 26
H = 64
NOBJ = 1024
K = 20
NACT = 8

AH = A * H
BN = 512
GN = N // BN
NFULL = N + NACT + 1
NW = 32
OBJ_PER_W = NOBJ // NW
IDX_PER_W = OBJ_PER_W * K
CHUNK = 128
NCHUNK = IDX_PER_W // CHUNK

_CONTRACT0 = (((0,), (0,)), ((), ()))


def _entity_tile(valsT_ref, wf_ref, bf_ref, q_ref, wa_ref, ba_ref, went_ref,
                 atnT_ref, outT_ref, el2_ref, rtq_s):
    i = pl.program_id(0)

    @pl.when(i == 0)
    def _():
        eye = jnp.eye(A, dtype=jnp.float32)
        rtq_s[...] = (eye[:, None, :] * q_ref[...][None, :, :]).reshape(AH, A)

    vt = valsT_ref[...]
    v_rep = jnp.broadcast_to(vt[:, None, :], (A, H, BN)).reshape(AH, BN)
    th = jnp.tanh(v_rep * wf_ref[...] + bf_ref[...])
    scoresT = lax.dot_general(rtq_s[...], th, _CONTRACT0,
                              preferred_element_type=jnp.float32)
    m = jnp.max(scoresT, axis=0, keepdims=True)
    e = jnp.exp(scoresT - m)
    wT = e / jnp.sum(e, axis=0, keepdims=True)
    elT = (lax.dot_general(wa_ref[...], wT * vt, _CONTRACT0,
                           preferred_element_type=jnp.float32)
           + lax.dot_general(ba_ref[...], wT, _CONTRACT0,
                             preferred_element_type=jnp.float32))
    outT_ref[...] = elT
    el2 = lax.dot_general(elT, went_ref[...], _CONTRACT0,
                          preferred_element_type=jnp.float32)
    el2_ref[...] = jnp.concatenate([el2, jnp.zeros_like(el2)], axis=1)

    @pl.when(i == GN)
    def _():
        outT_ref[:, 0:16] = atnT_ref[...]


def _entity_lookup(attr_vals, W_attr, b_attr, q_attn, W_ent, action_table):
    valsT = attr_vals.T
    wf = W_attr.reshape(AH, 1)
    bf = b_attr.reshape(AH, 1)
    qc = q_attn.reshape(H, 1)
    atnT = jnp.concatenate(
        [action_table.T, jnp.zeros((H, 16 - NACT), jnp.float32)], axis=1)
    clamp = lambda i: (0, jnp.minimum(i, GN - 1))
    full = lambda *shape: pl.BlockSpec(shape, lambda i: (0,) * len(shape))
    return pl.pallas_call(
        _entity_tile,
        grid=(GN + 1,),
        in_specs=[
            pl.BlockSpec((A, BN), clamp),
            full(AH, 1), full(AH, 1), full(H, 1),
            full(A, H), full(A, H), full(H, H), full(H, 16),
        ],
        out_specs=[
            pl.BlockSpec((H, BN), lambda i: (0, i)),
            pl.BlockSpec((BN, 2 * H), lambda i: (jnp.minimum(i, GN - 1), 0)),
        ],
        out_shape=[
            jax.ShapeDtypeStruct((H, NFULL), jnp.float32),
            jax.ShapeDtypeStruct((N, 2 * H), jnp.float32),
        ],
        scratch_shapes=[pltpu.VMEM((AH, A), jnp.float32)],
    )(valsT, wf, bf, qc, W_attr, b_attr, W_ent, atnT)


def _gather_body(el2_hbm, idx_hbm, out_hbm, idx_v, rows_v, pooled_v, sem):
    wid = lax.axis_index("s") * 2 + lax.axis_index("c")
    pltpu.sync_copy(idx_hbm.at[wid], idx_v)
    copies = [
        pltpu.async_copy(el2_hbm.at[idx_v.at[j]],
                         rows_v.at[pl.ds(j * CHUNK, CHUNK)], sem)
        for j in range(NCHUNK)
    ]
    for c in copies:
        c.wait()

    inv_k = jnp.float32(1.0 / K)

    def body(o, carry):
        for c in range(H // 16):
            acc = rows_v[o * K, pl.ds(c * 16, 16)]
            for k in range(1, K):
                acc = acc + rows_v[o * K + k, pl.ds(c * 16, 16)]
            y = acc * inv_k
            t = 1.0 - 2.0 / (jnp.exp(2.0 * y) + 1.0)
            pooled_v[o, pl.ds(c * 16, 16)] = t
        return carry

    lax.fori_loop(0, OBJ_PER_W, body, 0, unroll=False)
    pltpu.sync_copy(pooled_v, out_hbm.at[pl.ds(wid * OBJ_PER_W, OBJ_PER_W)])


def _gather_pool(el2, idx3d):
    mesh = plsc.VectorSubcoreMesh(core_axis_name="c", subcore_axis_name="s")
    fn = functools.partial(
        pl.kernel,
        mesh=mesh,
        out_type=jax.ShapeDtypeStruct((NOBJ, H), jnp.float32),
        scratch_types=[
            pltpu.VMEM((NCHUNK, CHUNK), jnp.int32),
            pltpu.VMEM((IDX_PER_W, 2 * H), jnp.float32),
            pltpu.VMEM((OBJ_PER_W, H), jnp.float32),
            pltpu.SemaphoreType.DMA,
        ],
    )(_gather_body)
    return fn(el2, idx3d)


def kernel(attr_vals, obs_idx, W_attr, b_attr, q_attn, W_ent, action_table):
    el_fullT, el2 = _entity_lookup(attr_vals, W_attr, b_attr, q_attn,
                                   W_ent, action_table)
    idx3d = obs_idx.astype(jnp.int32).reshape(NW, NCHUNK, CHUNK)
    obs = _gather_pool(el2, idx3d)
    return obs, el_fullT.T


# speedup vs baseline: 3.9216x; 1.2029x over previous
import functools

import jax
import jax.numpy as jnp
from jax import lax
from jax.experimental import pallas as pl
from jax.experimental.pallas import tpu as pltpu
from jax.experimental.pallas import tpu_sc as plsc

N = 16384
A = 26
H = 64
NOBJ = 1024
K = 20
NACT = 8

AH = A * H
BN = 1024
GN = N // BN
NFULL = N + NACT + 1
NW = 32
OBJ_PER_W = NOBJ // NW
IDX_PER_W = OBJ_PER_W * K
CHUNK = 128
NCHUNK = IDX_PER_W // CHUNK

_CONTRACT0 = (((0,), (0,)), ((), ()))


def _entity_tile(valsT_ref, wf_ref, bf_ref, q_ref, wa_ref, ba_ref, went_ref,
                 atnT_ref, outT_ref, el2_ref, rtq_s):
    i = pl.program_id(0)

    @pl.when(i == 0)
    def _():
        eye = jnp.eye(A, dtype=jnp.float32)
        rtq_s[...] = (eye[:, None, :] * q_ref[...][None, :, :]).reshape(AH, A)

    vt = valsT_ref[...]
    v_rep = jnp.broadcast_to(vt[:, None, :], (A, H, BN)).reshape(AH, BN)
    th = jnp.tanh(v_rep * wf_ref[...] + bf_ref[...])
    scoresT = lax.dot_general(rtq_s[...], th, _CONTRACT0,
                              preferred_element_type=jnp.float32)
    m = jnp.max(scoresT, axis=0, keepdims=True)
    e = jnp.exp(scoresT - m)
    wT = e / jnp.sum(e, axis=0, keepdims=True)
    elT = (lax.dot_general(wa_ref[...], wT * vt, _CONTRACT0,
                           preferred_element_type=jnp.float32)
           + lax.dot_general(ba_ref[...], wT, _CONTRACT0,
                             preferred_element_type=jnp.float32))
    outT_ref[...] = elT
    el2 = lax.dot_general(elT, went_ref[...], _CONTRACT0,
                          preferred_element_type=jnp.float32)
    el2_ref[...] = jnp.concatenate([el2, jnp.zeros_like(el2)], axis=1)

    @pl.when(i == GN)
    def _():
        outT_ref[:, 0:16] = atnT_ref[...]


def _entity_lookup(attr_vals, W_attr, b_attr, q_attn, W_ent, action_table):
    valsT = attr_vals.T
    wf = W_attr.reshape(AH, 1)
    bf = b_attr.reshape(AH, 1)
    qc = q_attn.reshape(H, 1)
    atnT = jnp.concatenate(
        [action_table.T, jnp.zeros((H, 16 - NACT), jnp.float32)], axis=1)
    clamp = lambda i: (0, jnp.minimum(i, GN - 1))
    full = lambda *shape: pl.BlockSpec(shape, lambda i: (0,) * len(shape))
    return pl.pallas_call(
        _entity_tile,
        grid=(GN + 1,),
        in_specs=[
            pl.BlockSpec((A, BN), clamp),
            full(AH, 1), full(AH, 1), full(H, 1),
            full(A, H), full(A, H), full(H, H), full(H, 16),
        ],
        out_specs=[
            pl.BlockSpec((H, BN), lambda i: (0, i)),
            pl.BlockSpec((BN, 2 * H), lambda i: (jnp.minimum(i, GN - 1), 0)),
        ],
        out_shape=[
            jax.ShapeDtypeStruct((H, NFULL), jnp.float32),
            jax.ShapeDtypeStruct((N, 2 * H), jnp.float32),
        ],
        scratch_shapes=[pltpu.VMEM((AH, A), jnp.float32)],
    )(valsT, wf, bf, qc, W_attr, b_attr, W_ent, atnT)


def _gather_body(el2_hbm, idx_hbm, out_hbm, idx_v, rows_v, pooled_v, sem):
    wid = lax.axis_index("s") * 2 + lax.axis_index("c")
    pltpu.sync_copy(idx_hbm.at[wid], idx_v)
    copies = [
        pltpu.async_copy(el2_hbm.at[idx_v.at[j]],
                         rows_v.at[pl.ds(j * CHUNK, CHUNK)], sem)
        for j in range(NCHUNK)
    ]
    for c in copies:
        c.wait()

    inv_k = jnp.float32(1.0 / K)

    def body(o, carry):
        for c in range(H // 16):
            acc = rows_v[o * K, pl.ds(c * 16, 16)]
            for k in range(1, K):
                acc = acc + rows_v[o * K + k, pl.ds(c * 16, 16)]
            y = acc * inv_k
            t = 1.0 - 2.0 / (jnp.exp(2.0 * y) + 1.0)
            pooled_v[o, pl.ds(c * 16, 16)] = t
        return carry

    lax.fori_loop(0, OBJ_PER_W, body, 0, unroll=False)
    pltpu.sync_copy(pooled_v, out_hbm.at[pl.ds(wid * OBJ_PER_W, OBJ_PER_W)])


def _gather_pool(el2, idx3d):
    mesh = plsc.VectorSubcoreMesh(core_axis_name="c", subcore_axis_name="s")
    fn = functools.partial(
        pl.kernel,
        mesh=mesh,
        out_type=jax.ShapeDtypeStruct((NOBJ, H), jnp.float32),
        scratch_types=[
            pltpu.VMEM((NCHUNK, CHUNK), jnp.int32),
            pltpu.VMEM((IDX_PER_W, 2 * H), jnp.float32),
            pltpu.VMEM((OBJ_PER_W, H), jnp.float32),
            pltpu.SemaphoreType.DMA,
        ],
    )(_gather_body)
    return fn(el2, idx3d)


def kernel(attr_vals, obs_idx, W_attr, b_attr, q_attn, W_ent, action_table):
    el_fullT, el2 = _entity_lookup(attr_vals, W_attr, b_attr, q_attn,
                                   W_ent, action_table)
    idx3d = obs_idx.astype(jnp.int32).reshape(NW, NCHUNK, CHUNK)
    obs = _gather_pool(el2, idx3d)
    return obs, el_fullT.T


# speedup vs baseline: 4.1820x; 1.0664x over previous
import functools

import jax
import jax.numpy as jnp
from jax import lax
from jax.experimental import pallas as pl
from jax.experimental.pallas import tpu as pltpu
from jax.experimental.pallas import tpu_sc as plsc

N = 16384
A = 26
H = 64
NOBJ = 1024
K = 20
NACT = 8

AH = A * H
BN = 2048
GN = N // BN
NFULL = N + NACT + 1
NW = 32
OBJ_PER_W = NOBJ // NW
IDX_PER_W = OBJ_PER_W * K
CHUNK = 128
NCHUNK = IDX_PER_W // CHUNK

_CONTRACT0 = (((0,), (0,)), ((), ()))


def _entity_tile(valsT_ref, wf_ref, bf_ref, q_ref, wa_ref, ba_ref, went_ref,
                 atnT_ref, outT_ref, el2_ref, rtq_s):
    i = pl.program_id(0)

    @pl.when(i == 0)
    def _():
        eye = jnp.eye(A, dtype=jnp.float32)
        rtq_s[...] = (eye[:, None, :] * q_ref[...][None, :, :]).reshape(AH, A)

    vt = valsT_ref[...]
    v_rep = jnp.broadcast_to(vt[:, None, :], (A, H, BN)).reshape(AH, BN)
    th = jnp.tanh(v_rep * wf_ref[...] + bf_ref[...])
    scoresT = lax.dot_general(rtq_s[...], th, _CONTRACT0,
                              preferred_element_type=jnp.float32)
    m = jnp.max(scoresT, axis=0, keepdims=True)
    e = jnp.exp(scoresT - m)
    wT = e / jnp.sum(e, axis=0, keepdims=True)
    elT = (lax.dot_general(wa_ref[...], wT * vt, _CONTRACT0,
                           preferred_element_type=jnp.float32)
           + lax.dot_general(ba_ref[...], wT, _CONTRACT0,
                             preferred_element_type=jnp.float32))
    outT_ref[...] = elT
    el2 = lax.dot_general(elT, went_ref[...], _CONTRACT0,
                          preferred_element_type=jnp.float32)
    el2_ref[...] = jnp.concatenate([el2, jnp.zeros_like(el2)], axis=1)

    @pl.when(i == GN)
    def _():
        outT_ref[:, 0:16] = atnT_ref[...]


def _entity_lookup(attr_vals, W_attr, b_attr, q_attn, W_ent, action_table):
    valsT = attr_vals.T
    wf = W_attr.reshape(AH, 1)
    bf = b_attr.reshape(AH, 1)
    qc = q_attn.reshape(H, 1)
    atnT = jnp.concatenate(
        [action_table.T, jnp.zeros((H, 16 - NACT), jnp.float32)], axis=1)
    clamp = lambda i: (0, jnp.minimum(i, GN - 1))
    full = lambda *shape: pl.BlockSpec(shape, lambda i: (0,) * len(shape))
    return pl.pallas_call(
        _entity_tile,
        grid=(GN + 1,),
        in_specs=[
            pl.BlockSpec((A, BN), clamp),
            full(AH, 1), full(AH, 1), full(H, 1),
            full(A, H), full(A, H), full(H, H), full(H, 16),
        ],
        out_specs=[
            pl.BlockSpec((H, BN), lambda i: (0, i)),
            pl.BlockSpec((BN, 2 * H), lambda i: (jnp.minimum(i, GN - 1), 0)),
        ],
        out_shape=[
            jax.ShapeDtypeStruct((H, NFULL), jnp.float32),
            jax.ShapeDtypeStruct((N, 2 * H), jnp.float32),
        ],
        scratch_shapes=[pltpu.VMEM((AH, A), jnp.float32)],
    )(valsT, wf, bf, qc, W_attr, b_attr, W_ent, atnT)


def _gather_body(el2_hbm, idx_hbm, out_hbm, idx_v, rows_v, pooled_v, sem):
    wid = lax.axis_index("s") * 2 + lax.axis_index("c")
    pltpu.sync_copy(idx_hbm.at[wid], idx_v)
    copies = [
        pltpu.async_copy(el2_hbm.at[idx_v.at[j]],
                         rows_v.at[pl.ds(j * CHUNK, CHUNK)], sem)
        for j in range(NCHUNK)
    ]
    for c in copies:
        c.wait()

    inv_k = jnp.float32(1.0 / K)

    def body(o, carry):
        for c in range(H // 16):
            acc = rows_v[o * K, pl.ds(c * 16, 16)]
            for k in range(1, K):
                acc = acc + rows_v[o * K + k, pl.ds(c * 16, 16)]
            y = acc * inv_k
            t = 1.0 - 2.0 / (jnp.exp(2.0 * y) + 1.0)
            pooled_v[o, pl.ds(c * 16, 16)] = t
        return carry

    lax.fori_loop(0, OBJ_PER_W, body, 0, unroll=False)
    pltpu.sync_copy(pooled_v, out_hbm.at[pl.ds(wid * OBJ_PER_W, OBJ_PER_W)])


def _gather_pool(el2, idx3d):
    mesh = plsc.VectorSubcoreMesh(core_axis_name="c", subcore_axis_name="s")
    fn = functools.partial(
        pl.kernel,
        mesh=mesh,
        out_type=jax.ShapeDtypeStruct((NOBJ, H), jnp.float32),
        scratch_types=[
            pltpu.VMEM((NCHUNK, CHUNK), jnp.int32),
            pltpu.VMEM((IDX_PER_W, 2 * H), jnp.float32),
            pltpu.VMEM((OBJ_PER_W, H), jnp.float32),
            pltpu.SemaphoreType.DMA,
        ],
    )(_gather_body)
    return fn(el2, idx3d)


def kernel(attr_vals, obs_idx, W_attr, b_attr, q_attn, W_ent, action_table):
    el_fullT, el2 = _entity_lookup(attr_vals, W_attr, b_attr, q_attn,
                                   W_ent, action_table)
    idx3d = obs_idx.astype(jnp.int32).reshape(NW, NCHUNK, CHUNK)
    obs = _gather_pool(el2, idx3d)
    return obs, el_fullT.T


# speedup vs baseline: 4.2192x; 1.0089x over previous
import functools

import jax
import jax.numpy as jnp
from jax import lax
from jax.experimental import pallas as pl
from jax.experimental.pallas import tpu as pltpu
from jax.experimental.pallas import tpu_sc as plsc

N = 16384
A = 26
H = 64
NOBJ = 1024
K = 20
NACT = 8

AH = A * H
BN = 2048
GN = N // BN
NFULL = N + NACT + 1
NW = 32
OBJ_PER_W = NOBJ // NW
IDX_PER_W = OBJ_PER_W * K
CHUNK = 128
NCHUNK = IDX_PER_W // CHUNK

_CONTRACT0 = (((0,), (0,)), ((), ()))


def _entity_tile(valsT_ref, wf_ref, bf_ref, q_ref, wa_ref, ba_ref, went_ref,
                 atnT_ref, outT_ref, el2_ref, rtq_s):
    i = pl.program_id(0)

    @pl.when(i == 0)
    def _():
        eye = jnp.eye(A, dtype=jnp.float32)
        rtq_s[...] = (eye[:, None, :]
                      * q_ref[...][None, :, :]).reshape(AH, A).astype(jnp.bfloat16)

    vt = valsT_ref[...]
    vtb = vt.astype(jnp.bfloat16)
    v_rep = jnp.broadcast_to(vtb[:, None, :], (A, H, BN)).reshape(AH, BN)
    th = jnp.tanh(v_rep * wf_ref[...].astype(jnp.bfloat16)
                  + bf_ref[...].astype(jnp.bfloat16))
    scoresT = lax.dot_general(rtq_s[...], th, _CONTRACT0,
                              preferred_element_type=jnp.float32)
    m = jnp.max(scoresT, axis=0, keepdims=True)
    e = jnp.exp(scoresT - m)
    wT = e / jnp.sum(e, axis=0, keepdims=True)
    elT = (lax.dot_general(wa_ref[...], wT * vt, _CONTRACT0,
                           preferred_element_type=jnp.float32)
           + lax.dot_general(ba_ref[...], wT, _CONTRACT0,
                             preferred_element_type=jnp.float32))
    outT_ref[...] = elT
    el2 = lax.dot_general(elT, went_ref[...], _CONTRACT0,
                          preferred_element_type=jnp.float32)
    el2_ref[...] = jnp.concatenate([el2, jnp.zeros_like(el2)], axis=1)

    @pl.when(i == GN)
    def _():
        outT_ref[:, 0:16] = atnT_ref[...]


def _entity_lookup(attr_vals, W_attr, b_attr, q_attn, W_ent, action_table):
    valsT = attr_vals.T
    wf = W_attr.reshape(AH, 1)
    bf = b_attr.reshape(AH, 1)
    qc = q_attn.reshape(H, 1)
    atnT = jnp.concatenate(
        [action_table.T, jnp.zeros((H, 16 - NACT), jnp.float32)], axis=1)
    clamp = lambda i: (0, jnp.minimum(i, GN - 1))
    full = lambda *shape: pl.BlockSpec(shape, lambda i: (0,) * len(shape))
    return pl.pallas_call(
        _entity_tile,
        grid=(GN + 1,),
        in_specs=[
            pl.BlockSpec((A, BN), clamp),
            full(AH, 1), full(AH, 1), full(H, 1),
            full(A, H), full(A, H), full(H, H), full(H, 16),
        ],
        out_specs=[
            pl.BlockSpec((H, BN), lambda i: (0, i)),
            pl.BlockSpec((BN, 2 * H), lambda i: (jnp.minimum(i, GN - 1), 0)),
        ],
        out_shape=[
            jax.ShapeDtypeStruct((H, NFULL), jnp.float32),
            jax.ShapeDtypeStruct((N, 2 * H), jnp.float32),
        ],
        scratch_shapes=[pltpu.VMEM((AH, A), jnp.bfloat16)],
    )(valsT, wf, bf, qc, W_attr, b_attr, W_ent, atnT)


def _gather_body(el2_hbm, idx_hbm, out_hbm, idx_v, rows_v, pooled_v, sem):
    wid = lax.axis_index("s") * 2 + lax.axis_index("c")
    pltpu.sync_copy(idx_hbm.at[wid], idx_v)
    copies = [
        pltpu.async_copy(el2_hbm.at[idx_v.at[j]],
                         rows_v.at[pl.ds(j * CHUNK, CHUNK)], sem)
        for j in range(NCHUNK)
    ]
    for c in copies:
        c.wait()

    inv_k = jnp.float32(1.0 / K)

    def body(o, carry):
        for c in range(H // 16):
            acc = rows_v[o * K, pl.ds(c * 16, 16)]
            for k in range(1, K):
                acc = acc + rows_v[o * K + k, pl.ds(c * 16, 16)]
            y = acc * inv_k
            t = 1.0 - 2.0 / (jnp.exp(2.0 * y) + 1.0)
            pooled_v[o, pl.ds(c * 16, 16)] = t
        return carry

    lax.fori_loop(0, OBJ_PER_W, body, 0, unroll=False)
    pltpu.sync_copy(pooled_v, out_hbm.at[pl.ds(wid * OBJ_PER_W, OBJ_PER_W)])


def _gather_pool(el2, idx3d):
    mesh = plsc.VectorSubcoreMesh(core_axis_name="c", subcore_axis_name="s")
    fn = functools.partial(
        pl.kernel,
        mesh=mesh,
        out_type=jax.ShapeDtypeStruct((NOBJ, H), jnp.float32),
        scratch_types=[
            pltpu.VMEM((NCHUNK, CHUNK), jnp.int32),
            pltpu.VMEM((IDX_PER_W, 2 * H), jnp.float32),
            pltpu.VMEM((OBJ_PER_W, H), jnp.float32),
            pltpu.SemaphoreType.DMA,
        ],
    )(_gather_body)
    return fn(el2, idx3d)


def kernel(attr_vals, obs_idx, W_attr, b_attr, q_attn, W_ent, action_table):
    el_fullT, el2 = _entity_lookup(attr_vals, W_attr, b_attr, q_attn,
                                   W_ent, action_table)
    idx3d = obs_idx.astype(jnp.int32).reshape(NW, NCHUNK, CHUNK)
    obs = _gather_pool(el2, idx3d)
    return obs, el_fullT.T
